# Initial kernel scaffold; baseline (speedup 1.0000x reference)
import jax
import jax.numpy as jnp
from jax.experimental import pallas as pl


def kernel(x, edge_index, W_self, W_neigh, bias):
    raise NotImplementedError("write your pallas kernel here")


# speedup vs baseline: 3.9069x; 3.9069x over previous
import jax
import jax.numpy as jnp
from jax import lax
from jax.experimental import pallas as pl
from jax.experimental.pallas import tpu as pltpu
from jax.experimental.pallas import tpu_sc as plsc

N_NODES = 10000
N_EDGES = 320000
D = 128

NC = 2
NS = 16
NW = NC * NS
CHUNK = 128

N_PAD = ((N_NODES + NS * CHUNK - 1) // (NS * CHUNK)) * (NS * CHUNK)
CPW = -(-((N_EDGES + NW * CHUNK - 1) // (NW * CHUNK)) // 8) * 8
EPW = CPW * CHUNK
E_PAD = NW * EPW
ROWS_PER_TILE = N_PAD // NS


def _sc_body(src2d, dst2d, x_hbm, summed_hbm, deg_hbm,
             idx_src_v, idx_dst_v, rows_v, deg_v, summed_sh, sem):
    c = lax.axis_index("c")
    s = lax.axis_index("s")
    wid = s * NC + c

    pltpu.sync_copy(src2d.at[pl.ds(wid * CPW, CPW)], idx_src_v)
    pltpu.sync_copy(dst2d.at[pl.ds(wid * CPW, CPW)], idx_dst_v)

    zeros16 = jnp.zeros((16,), jnp.float32)
    ones16 = jnp.ones((16,), jnp.float32)

    def _zdeg(i, _):
        deg_v[pl.ds(i * 16, 16)] = zeros16
        return 0
    lax.fori_loop(0, N_PAD // 16, _zdeg, 0)

    def _zrow(i, _):
        rows_v[i // (D // 16), pl.ds((i % (D // 16)) * 16, 16)] = zeros16
        return 0
    lax.fori_loop(0, CHUNK * (D // 16), _zrow, 0)

    def _zsh(k, _):
        pltpu.sync_copy(rows_v, summed_sh.at[pl.ds(s * ROWS_PER_TILE + k * CHUNK, CHUNK)])
        return 0
    lax.fori_loop(0, ROWS_PER_TILE // CHUNK, _zsh, 0)

    plsc.subcore_barrier()

    def _edge(i, _):
        pltpu.async_copy(x_hbm.at[idx_src_v.at[i]], rows_v, sem).wait()
        pltpu.sync_copy(rows_v, summed_sh.at[idx_dst_v.at[i]], add=True)
        return 0
    lax.fori_loop(0, CPW, _edge, 0)

    def _deg(i, _):
        for j in range(CHUNK // 16):
            idx16 = idx_dst_v[i, pl.ds(j * 16, 16)]
            plsc.addupdate_scatter(deg_v, [idx16], ones16)
        return 0
    lax.fori_loop(0, CPW, _deg, 0)

    pltpu.sync_copy(deg_v, deg_hbm.at[wid])

    plsc.subcore_barrier()

    def _dump(k, _):
        r0 = s * ROWS_PER_TILE + k * CHUNK
        pltpu.sync_copy(summed_sh.at[pl.ds(r0, CHUNK)],
                        summed_hbm.at[pl.ds(c * N_PAD + r0, CHUNK)])
        return 0
    lax.fori_loop(0, ROWS_PER_TILE // CHUNK, _dump, 0)


_sc_scatter = pl.kernel(
    _sc_body,
    out_type=(
        jax.ShapeDtypeStruct((NC * N_PAD, D), jnp.float32),
        jax.ShapeDtypeStruct((NW, N_PAD), jnp.float32),
    ),
    mesh=plsc.VectorSubcoreMesh(core_axis_name="c", subcore_axis_name="s"),
    scratch_types=(
        pltpu.VMEM((CPW, CHUNK), jnp.int32),
        pltpu.VMEM((CPW, CHUNK), jnp.int32),
        pltpu.VMEM((CHUNK, D), jnp.float32),
        pltpu.VMEM((N_PAD,), jnp.float32),
        pltpu.VMEM_SHARED((N_PAD, D), jnp.float32),
        pltpu.SemaphoreType.DMA,
    ),
    compiler_params=pltpu.CompilerParams(needs_layout_passes=False),
)


def _tc_body(x_ref, summed_ref, degt_ref, wst_ref, wnt_ref, bias_ref, out_ref):
    ssum = summed_ref[:N_PAD, :] + summed_ref[N_PAD:, :]
    deg = jnp.sum(degt_ref[...], axis=1, keepdims=True)
    neigh = ssum * (1.0 / jnp.maximum(deg, 1.0))
    out_ref[...] = (
        jnp.dot(x_ref[...], wst_ref[...], preferred_element_type=jnp.float32)
        + jnp.dot(neigh, wnt_ref[...], preferred_element_type=jnp.float32)
        + bias_ref[...]
    )


_tc_epilogue = pl.pallas_call(
    _tc_body,
    out_shape=jax.ShapeDtypeStruct((N_PAD, D), jnp.float32),
)


def kernel(x, edge_index, W_self, W_neigh, bias):
    src = edge_index[0].astype(jnp.int32)
    dst = edge_index[1].astype(jnp.int32)
    x_pad = jnp.pad(x, ((0, N_PAD - N_NODES), (0, 0)))
    src_p = jnp.pad(src, (0, E_PAD - N_EDGES), constant_values=N_NODES)
    dst_p = jnp.pad(dst, (0, E_PAD - N_EDGES), constant_values=N_NODES)
    src2d = src_p.reshape(NW * CPW, CHUNK)
    dst2d = dst_p.reshape(NW * CPW, CHUNK)

    summed, deg = _sc_scatter(src2d, dst2d, x_pad)
    out = _tc_epilogue(x_pad, summed, deg.T, W_self.T, W_neigh.T,
                       bias.reshape(1, D))
    return out[:N_NODES]


# speedup vs baseline: 5.2806x; 1.3516x over previous
import jax
import jax.numpy as jnp
from jax import lax
from jax.experimental import pallas as pl
from jax.experimental.pallas import tpu as pltpu
from jax.experimental.pallas import tpu_sc as plsc

N_NODES = 10000
N_EDGES = 320000
D = 128

NC = 2
NS = 16
NW = NC * NS
CHUNK = 128
NBUF = 2
RB = 4

N_PAD = ((N_NODES + NS * CHUNK - 1) // (NS * CHUNK)) * (NS * CHUNK)
_CPW_MIN = (N_EDGES + NW * CHUNK - 1) // (NW * CHUNK)
CPW = -(-_CPW_MIN // RB) * RB
EPW = CPW * CHUNK
E_PAD = NW * EPW
ROWS_PER_TILE = N_PAD // NS


def _sc_body(srcf, dstf, x_hbm, summed_hbm, deg_hbm,
             isrc_v, idst_v, rows_v, deg_v, summed_sh,
             is0, is1, is2, is3, id0, id1, id2, id3, g0, g1, s0, s1):
    isems = (is0, is1, is2, is3)
    idsems = (id0, id1, id2, id3)
    gsems = (g0, g1)
    ssems = (s0, s1)

    c = lax.axis_index("c")
    s = lax.axis_index("s")
    wid = s * NC + c
    ebase = wid * EPW

    zeros16 = jnp.zeros((16,), jnp.float32)
    ones16 = jnp.ones((16,), jnp.float32)

    def _zdeg(i, _):
        deg_v[pl.ds(i * 16, 16)] = zeros16
        return 0
    lax.fori_loop(0, N_PAD // 16, _zdeg, 0)

    def _zrow(i, _):
        rows_v[0, i // (D // 16), pl.ds((i % (D // 16)) * 16, 16)] = zeros16
        return 0
    lax.fori_loop(0, CHUNK * (D // 16), _zrow, 0)

    def _zsh(k, _):
        pltpu.sync_copy(rows_v.at[0],
                        summed_sh.at[pl.ds(s * ROWS_PER_TILE + k * CHUNK, CHUNK)])
        return 0
    lax.fori_loop(0, ROWS_PER_TILE // CHUNK, _zsh, 0)

    plsc.subcore_barrier()

    def _ifire(i, slot):
        off = ebase + i * CHUNK
        pltpu.async_copy(srcf.at[pl.ds(off, CHUNK)], isrc_v.at[slot], isems[slot])
        pltpu.async_copy(dstf.at[pl.ds(off, CHUNK)], idst_v.at[slot], idsems[slot])

    def _iwait_src(slot):
        pltpu.make_async_copy(srcf.at[pl.ds(0, CHUNK)], isrc_v.at[slot],
                              isems[slot]).wait()

    def _iwait_dst(slot):
        pltpu.make_async_copy(dstf.at[pl.ds(0, CHUNK)], idst_v.at[slot],
                              idsems[slot]).wait()

    def _gfire(slot, b):
        pltpu.async_copy(x_hbm.at[isrc_v.at[slot]], rows_v.at[b], gsems[b])

    def _gwait(b):
        pltpu.make_async_copy(x_hbm.at[isrc_v.at[0]], rows_v.at[b],
                              gsems[b]).wait()

    def _sfire(slot, b):
        pltpu.async_copy(rows_v.at[b], summed_sh.at[idst_v.at[slot]],
                         ssems[b], add=True)

    def _swait(b):
        pltpu.make_async_copy(rows_v.at[b], summed_sh.at[idst_v.at[0]],
                              ssems[b]).wait()

    def _count_deg(slot):
        for j in range(CHUNK // 16):
            idx16 = idst_v[slot, pl.ds(j * 16, 16)]
            plsc.addupdate_scatter(deg_v, [idx16], ones16)

    for b in range(RB):
        _ifire(b, b)
    for b in range(NBUF):
        _iwait_src(b)
        _gfire(b, b)

    def _edge(t, _):
        for b in range(RB):
            i = t * RB + b
            _iwait_dst(b)
            _gwait(b % NBUF)
            _sfire(b, b % NBUF)
            _count_deg(b)
            _swait(b % NBUF)
            _iwait_src((b + NBUF) % RB)
            _gfire((b + NBUF) % RB, b % NBUF)
            _ifire(i + RB, b)
        return 0
    lax.fori_loop(0, CPW // RB - 1, _edge, 0)

    for b in range(RB):
        _iwait_dst(b)
        _gwait(b % NBUF)
        _sfire(b, b % NBUF)
        _count_deg(b)
        _swait(b % NBUF)
        if b < RB - NBUF:
            _iwait_src((b + NBUF) % RB)
            _gfire((b + NBUF) % RB, b % NBUF)

    pltpu.sync_copy(deg_v, deg_hbm.at[wid])

    plsc.subcore_barrier()

    def _dump(k, _):
        r0 = s * ROWS_PER_TILE + k * CHUNK
        pltpu.sync_copy(summed_sh.at[pl.ds(r0, CHUNK)],
                        summed_hbm.at[pl.ds(c * N_PAD + r0, CHUNK)])
        return 0
    lax.fori_loop(0, ROWS_PER_TILE // CHUNK, _dump, 0)


_sc_scatter = pl.kernel(
    _sc_body,
    out_type=(
        jax.ShapeDtypeStruct((NC * N_PAD, D), jnp.float32),
        jax.ShapeDtypeStruct((NW, N_PAD), jnp.float32),
    ),
    mesh=plsc.VectorSubcoreMesh(core_axis_name="c", subcore_axis_name="s"),
    scratch_types=(
        pltpu.VMEM((RB, CHUNK), jnp.int32),
        pltpu.VMEM((RB, CHUNK), jnp.int32),
        pltpu.VMEM((NBUF, CHUNK, D), jnp.float32),
        pltpu.VMEM((N_PAD,), jnp.float32),
        pltpu.VMEM_SHARED((N_PAD, D), jnp.float32),
    ) + (pltpu.SemaphoreType.DMA,) * 12,
    compiler_params=pltpu.CompilerParams(needs_layout_passes=False),
)


def _tc_body(x_ref, summed_ref, degt_ref, wst_ref, wnt_ref, bias_ref, out_ref):
    ssum = summed_ref[:N_PAD, :] + summed_ref[N_PAD:, :]
    deg = jnp.sum(degt_ref[...], axis=1, keepdims=True)
    neigh = ssum * (1.0 / jnp.maximum(deg, 1.0))
    out_ref[...] = (
        jnp.dot(x_ref[...], wst_ref[...], preferred_element_type=jnp.float32)
        + jnp.dot(neigh, wnt_ref[...], preferred_element_type=jnp.float32)
        + bias_ref[...]
    )


_tc_epilogue = pl.pallas_call(
    _tc_body,
    out_shape=jax.ShapeDtypeStruct((N_PAD, D), jnp.float32),
)


def kernel(x, edge_index, W_self, W_neigh, bias):
    src = edge_index[0].astype(jnp.int32)
    dst = edge_index[1].astype(jnp.int32)
    x_pad = jnp.pad(x, ((0, N_PAD - N_NODES), (0, 0)))
    src_p = jnp.pad(src, (0, E_PAD - N_EDGES), constant_values=N_NODES)
    dst_p = jnp.pad(dst, (0, E_PAD - N_EDGES), constant_values=N_NODES)

    summed, deg = _sc_scatter(src_p, dst_p, x_pad)
    out = _tc_epilogue(x_pad, summed, deg.T, W_self.T, W_neigh.T,
                       bias.reshape(1, D))
    return out[:N_NODES]
